# Initial kernel scaffold; baseline (speedup 1.0000x reference)
import jax
import jax.numpy as jnp
from jax.experimental import pallas as pl


def kernel(x, x_cont, table, bias, w):
    raise NotImplementedError("write your pallas kernel here")


# speedup vs baseline: 1.4424x; 1.4424x over previous
import jax
import jax.numpy as jnp
from jax import lax
from jax.experimental import pallas as pl
from jax.experimental.pallas import tpu as pltpu
from jax.experimental.pallas import tpu_sc as plsc

_FIELD_DIM = 38461
_NUM_FIELDS = 26
_CONT = 13
_BATCH = 16384
_NUM_CORES = 2
_NUM_SUBCORES = 16
_NW = _NUM_CORES * _NUM_SUBCORES
_BPW = _BATCH // _NW
_LANES = 16
_CHUNK = 128
_PER_FIELD_CHUNKS = _BPW // _CHUNK
_NCHUNK = _NUM_FIELDS * _PER_FIELD_CHUNKS


def _sc_body(xt_h, xc_h, table_h, wb_h, out_h,
             xt_v, xc_v, idx_v, g_v, wb_v, out_v, sem):
    c = lax.axis_index("c")
    s = lax.axis_index("s")
    wid = s * _NUM_CORES + c
    base = wid * _BPW

    pltpu.sync_copy(xt_h.at[:, pl.ds(base, _BPW)], xt_v)
    pltpu.sync_copy(xc_h.at[:, pl.ds(base, _BPW)], xc_v)
    pltpu.sync_copy(wb_h, wb_v)

    @pl.loop(0, _NCHUNK)
    def _idx_loop(r):
        f = r // _PER_FIELD_CHUNKS
        off = (r % _PER_FIELD_CHUNKS) * _CHUNK
        fo = f * _FIELD_DIM
        for k in range(_CHUNK // _LANES):
            v = xt_v[f, pl.ds(off + k * _LANES, _LANES)]
            idx_v[r, pl.ds(k * _LANES, _LANES)] = v + fo

    @pl.loop(0, _NCHUNK)
    def _fire(r):
        f = r // _PER_FIELD_CHUNKS
        off = (r % _PER_FIELD_CHUNKS) * _CHUNK
        pltpu.make_async_copy(
            table_h.at[idx_v.at[r]], g_v.at[f, pl.ds(off, _CHUNK)], sem
        ).start()

    @pl.loop(0, _NCHUNK)
    def _drain(r):
        f = r // _PER_FIELD_CHUNKS
        off = (r % _PER_FIELD_CHUNKS) * _CHUNK
        pltpu.make_async_copy(
            table_h.at[idx_v.at[r]], g_v.at[f, pl.ds(off, _CHUNK)], sem
        ).wait()

    wb_vec = wb_v[...]
    w_s = [wb_vec[i] for i in range(_CONT)]
    bias_s = wb_vec[_CONT]

    @pl.loop(0, _BPW // _LANES)
    def _acc_loop(j):
        o = j * _LANES
        acc = jnp.full((_LANES,), bias_s, jnp.float32)
        for f in range(_NUM_FIELDS):
            acc = acc + g_v[f, pl.ds(o, _LANES)]
        for cc in range(_CONT):
            acc = acc + xc_v[cc, pl.ds(o, _LANES)] * w_s[cc]
        out_v[pl.ds(o, _LANES)] = acc

    pltpu.sync_copy(out_v, out_h.at[pl.ds(base, _BPW)])


def _make_kernel():
    mesh = plsc.VectorSubcoreMesh(core_axis_name="c", subcore_axis_name="s")
    return pl.kernel(
        _sc_body,
        out_type=jax.ShapeDtypeStruct((_BATCH,), jnp.float32),
        mesh=mesh,
        scratch_types=[
            pltpu.VMEM((_NUM_FIELDS, _BPW), jnp.int32),
            pltpu.VMEM((_CONT, _BPW), jnp.float32),
            pltpu.VMEM((_NCHUNK, _CHUNK), jnp.int32),
            pltpu.VMEM((_NUM_FIELDS, _BPW), jnp.float32),
            pltpu.VMEM((_LANES,), jnp.float32),
            pltpu.VMEM((_BPW,), jnp.float32),
            pltpu.SemaphoreType.DMA,
        ],
    )


_fm_linear_sc = _make_kernel()


@jax.jit
def kernel(x, x_cont, table, bias, w):
    xt = x.T
    xct = x_cont.T
    tab = table.reshape(-1)
    wb = jnp.concatenate(
        [w, bias, jnp.zeros((_LANES - _CONT - 1,), jnp.float32)]
    )
    out = _fm_linear_sc(xt, xct, tab, wb)
    return out.reshape(-1, 1)
